# baseline (speedup 1.0000x reference)
import functools

import jax
import jax.numpy as jnp
from jax import lax
from jax.experimental import pallas as pl
from jax.experimental.pallas import tpu as pltpu
from jax.experimental.pallas import tpu_sc as plsc

BATCH = 1000
FEAT = 12288
ROWS = 1024

_SC_INFO = plsc.get_sparse_core_info()
_NC = _SC_INFO.num_cores
_NS = _SC_INFO.num_subcores
_NW = _NC * _NS
PAD_B = 1024
_PER_W = PAD_B // _NW


def _sc_gather_body(table_hbm, idx_hbm, out_hbm, idx_v, rows_v, sem):
    wid = lax.axis_index("s") * _NC + lax.axis_index("c")
    base = wid * _PER_W
    pltpu.sync_copy(idx_hbm.at[pl.ds(base, _PER_W)], idx_v)
    pltpu.async_copy(table_hbm.at[idx_v], rows_v, sem).wait()
    pltpu.sync_copy(rows_v, out_hbm.at[pl.ds(base, _PER_W)])


_sc_gather = functools.partial(
    pl.kernel,
    out_type=jax.ShapeDtypeStruct((PAD_B,), jnp.float32),
    mesh=plsc.VectorSubcoreMesh(core_axis_name="c", subcore_axis_name="s"),
    scratch_types=[
        pltpu.VMEM((_PER_W,), jnp.int32),
        pltpu.VMEM((_PER_W,), jnp.float32),
        pltpu.SemaphoreType.DMA,
    ],
)(_sc_gather_body)


CH = 1024
NCHUNK = FEAT // CH
NBUF = 4


def _fma_body(ac_t_ref, x0_hbm, nz_hbm, out_hbm,
              x0_buf, nz_buf, out_buf, x0_sem, nz_sem, out_sem):
    def start_in(i, slot):
        pltpu.make_async_copy(
            x0_hbm.at[pl.ds(i * CH, CH), :], x0_buf.at[slot], x0_sem.at[slot]
        ).start()
        pltpu.make_async_copy(
            nz_hbm.at[pl.ds(i * CH, CH), :], nz_buf.at[slot], nz_sem.at[slot]
        ).start()

    def out_copy(i, slot):
        return pltpu.make_async_copy(
            out_buf.at[slot], out_hbm.at[pl.ds(i * CH, CH), :], out_sem.at[slot]
        )

    for s in range(NBUF):
        start_in(s, s)

    a = ac_t_ref[...]
    sa = jnp.sqrt(a)
    sb = jnp.sqrt(jnp.maximum(1.0 - a, 0.0))

    def body(i, carry):
        slot = lax.rem(i, NBUF)
        pltpu.make_async_copy(
            x0_hbm.at[pl.ds(i * CH, CH), :], x0_buf.at[slot], x0_sem.at[slot]
        ).wait()
        pltpu.make_async_copy(
            nz_hbm.at[pl.ds(i * CH, CH), :], nz_buf.at[slot], nz_sem.at[slot]
        ).wait()

        @pl.when(i >= NBUF)
        def _():
            out_copy(i - NBUF, slot).wait()

        ov = out_buf.at[slot]
        ov[...] = sa * x0_buf[slot] + sb * nz_buf[slot]
        out_copy(i, slot).start()

        @pl.when(i + NBUF < NCHUNK)
        def _():
            start_in(i + NBUF, slot)

        return carry

    lax.fori_loop(0, NCHUNK, body, 0)
    for k in range(NBUF):
        i = NCHUNK - NBUF + k
        out_copy(i, i % NBUF).wait()


def kernel(x_0, timesteps, noise, alphas_cumprod):
    orig_shape = x_0.shape
    x0r = x_0.reshape(BATCH, FEAT).T
    nzr = noise.reshape(BATCH, FEAT).T
    ts = timesteps.astype(jnp.int32)
    ac = alphas_cumprod.astype(jnp.float32)

    ts_pad = jnp.pad(ts, (0, PAD_B - BATCH))
    ac_pad = jnp.pad(ac, (0, PAD_B - BATCH))
    ac_t = _sc_gather(ac_pad, ts_pad)[:BATCH].reshape(1, BATCH)

    out = pl.pallas_call(
        _fma_body,
        in_specs=[
            pl.BlockSpec(memory_space=pltpu.VMEM),
            pl.BlockSpec(memory_space=pl.ANY),
            pl.BlockSpec(memory_space=pl.ANY),
        ],
        out_specs=pl.BlockSpec(memory_space=pl.ANY),
        out_shape=jax.ShapeDtypeStruct((FEAT, BATCH), jnp.float32),
        scratch_shapes=[
            pltpu.VMEM((NBUF, CH, BATCH), jnp.float32),
            pltpu.VMEM((NBUF, CH, BATCH), jnp.float32),
            pltpu.VMEM((NBUF, CH, BATCH), jnp.float32),
            pltpu.SemaphoreType.DMA((NBUF,)),
            pltpu.SemaphoreType.DMA((NBUF,)),
            pltpu.SemaphoreType.DMA((NBUF,)),
        ],
    )(ac_t, x0r, nzr)
    return out.T.reshape(orig_shape)


# speedup vs baseline: 1.0001x; 1.0001x over previous
import functools

import jax
import jax.numpy as jnp
from jax import lax
from jax.experimental import pallas as pl
from jax.experimental.pallas import tpu as pltpu
from jax.experimental.pallas import tpu_sc as plsc

BATCH = 1000
FEAT = 12288
ROWS = 1024

_SC_INFO = plsc.get_sparse_core_info()
_NC = _SC_INFO.num_cores
_NS = _SC_INFO.num_subcores
_NW = _NC * _NS
PAD_B = 1024
_PER_W = PAD_B // _NW


def _sc_gather_body(table_hbm, idx_hbm, out_hbm, idx_v, rows_v, sem):
    wid = lax.axis_index("s") * _NC + lax.axis_index("c")
    base = wid * _PER_W
    pltpu.sync_copy(idx_hbm.at[pl.ds(base, _PER_W)], idx_v)
    pltpu.async_copy(table_hbm.at[idx_v], rows_v, sem).wait()
    pltpu.sync_copy(rows_v, out_hbm.at[pl.ds(base, _PER_W)])


_sc_gather = functools.partial(
    pl.kernel,
    out_type=jax.ShapeDtypeStruct((PAD_B,), jnp.float32),
    mesh=plsc.VectorSubcoreMesh(core_axis_name="c", subcore_axis_name="s"),
    scratch_types=[
        pltpu.VMEM((_PER_W,), jnp.int32),
        pltpu.VMEM((_PER_W,), jnp.float32),
        pltpu.SemaphoreType.DMA,
    ],
)(_sc_gather_body)


CH = 256
NCHUNK = FEAT // CH
NBUF = 12


def _fma_body(ac_t_ref, x0_hbm, nz_hbm, out_hbm,
              x0_buf, nz_buf, out_buf, x0_sem, nz_sem, out_sem):
    def start_in(i, slot):
        pltpu.make_async_copy(
            x0_hbm.at[pl.ds(i * CH, CH), :], x0_buf.at[slot], x0_sem.at[slot]
        ).start()
        pltpu.make_async_copy(
            nz_hbm.at[pl.ds(i * CH, CH), :], nz_buf.at[slot], nz_sem.at[slot]
        ).start()

    def out_copy(i, slot):
        return pltpu.make_async_copy(
            out_buf.at[slot], out_hbm.at[pl.ds(i * CH, CH), :], out_sem.at[slot]
        )

    for s in range(NBUF):
        start_in(s, s)

    a = ac_t_ref[...]
    sa = jnp.sqrt(a)
    sb = jnp.sqrt(jnp.maximum(1.0 - a, 0.0))

    def body(i, carry):
        slot = lax.rem(i, NBUF)
        pltpu.make_async_copy(
            x0_hbm.at[pl.ds(i * CH, CH), :], x0_buf.at[slot], x0_sem.at[slot]
        ).wait()
        pltpu.make_async_copy(
            nz_hbm.at[pl.ds(i * CH, CH), :], nz_buf.at[slot], nz_sem.at[slot]
        ).wait()

        @pl.when(i >= NBUF)
        def _():
            out_copy(i - NBUF, slot).wait()

        ov = out_buf.at[slot]
        ov[...] = sa * x0_buf[slot] + sb * nz_buf[slot]
        out_copy(i, slot).start()

        @pl.when(i + NBUF < NCHUNK)
        def _():
            start_in(i + NBUF, slot)

        return carry

    lax.fori_loop(0, NCHUNK, body, 0)
    for k in range(NBUF):
        i = NCHUNK - NBUF + k
        out_copy(i, i % NBUF).wait()


def kernel(x_0, timesteps, noise, alphas_cumprod):
    orig_shape = x_0.shape
    x0r = x_0.reshape(BATCH, FEAT).T
    nzr = noise.reshape(BATCH, FEAT).T
    ts = timesteps.astype(jnp.int32)
    ac = alphas_cumprod.astype(jnp.float32)

    ts_pad = jnp.pad(ts, (0, PAD_B - BATCH))
    ac_pad = jnp.pad(ac, (0, PAD_B - BATCH))
    ac_t = _sc_gather(ac_pad, ts_pad)[:BATCH].reshape(1, BATCH)

    out = pl.pallas_call(
        _fma_body,
        in_specs=[
            pl.BlockSpec(memory_space=pltpu.VMEM),
            pl.BlockSpec(memory_space=pl.ANY),
            pl.BlockSpec(memory_space=pl.ANY),
        ],
        out_specs=pl.BlockSpec(memory_space=pl.ANY),
        out_shape=jax.ShapeDtypeStruct((FEAT, BATCH), jnp.float32),
        scratch_shapes=[
            pltpu.VMEM((NBUF, CH, BATCH), jnp.float32),
            pltpu.VMEM((NBUF, CH, BATCH), jnp.float32),
            pltpu.VMEM((NBUF, CH, BATCH), jnp.float32),
            pltpu.SemaphoreType.DMA((NBUF,)),
            pltpu.SemaphoreType.DMA((NBUF,)),
            pltpu.SemaphoreType.DMA((NBUF,)),
        ],
    )(ac_t, x0r, nzr)
    return out.T.reshape(orig_shape)


# speedup vs baseline: 1.0038x; 1.0037x over previous
import functools

import jax
import jax.numpy as jnp
from jax import lax
from jax.experimental import pallas as pl
from jax.experimental.pallas import tpu as pltpu
from jax.experimental.pallas import tpu_sc as plsc

BATCH = 1000
FEAT = 12288
ROWS = 1024

_SC_INFO = plsc.get_sparse_core_info()
_NC = _SC_INFO.num_cores
_NS = _SC_INFO.num_subcores
_NW = _NC * _NS
PAD_B = 1024
_PER_W = PAD_B // _NW


def _sc_gather_body(table_hbm, idx_hbm, out_hbm, idx_v, rows_v, sem):
    wid = lax.axis_index("s") * _NC + lax.axis_index("c")
    base = wid * _PER_W
    pltpu.sync_copy(idx_hbm.at[pl.ds(base, _PER_W)], idx_v)
    pltpu.async_copy(table_hbm.at[idx_v], rows_v, sem).wait()
    pltpu.sync_copy(rows_v, out_hbm.at[pl.ds(base, _PER_W)])


_sc_gather = functools.partial(
    pl.kernel,
    out_type=jax.ShapeDtypeStruct((PAD_B,), jnp.float32),
    mesh=plsc.VectorSubcoreMesh(core_axis_name="c", subcore_axis_name="s"),
    scratch_types=[
        pltpu.VMEM((_PER_W,), jnp.int32),
        pltpu.VMEM((_PER_W,), jnp.float32),
        pltpu.SemaphoreType.DMA,
    ],
)(_sc_gather_body)


CH = 512
NCHUNK = FEAT // CH
NBUF = 6


def _fma_body(ac_t_ref, x0_hbm, nz_hbm, out_hbm,
              x0_buf, nz_buf, out_buf, x0_sem, nz_sem, out_sem):
    H = CH // 2

    def start_in(i, slot):
        for h in range(2):
            pltpu.make_async_copy(
                x0_hbm.at[pl.ds(i * CH + h * H, H), :],
                x0_buf.at[slot, pl.ds(h * H, H)],
                x0_sem.at[slot],
            ).start()
        for h in range(2):
            pltpu.make_async_copy(
                nz_hbm.at[pl.ds(i * CH + h * H, H), :],
                nz_buf.at[slot, pl.ds(h * H, H)],
                nz_sem.at[slot],
            ).start()

    def wait_in(i, slot):
        for h in range(2):
            pltpu.make_async_copy(
                x0_hbm.at[pl.ds(i * CH + h * H, H), :],
                x0_buf.at[slot, pl.ds(h * H, H)],
                x0_sem.at[slot],
            ).wait()
        for h in range(2):
            pltpu.make_async_copy(
                nz_hbm.at[pl.ds(i * CH + h * H, H), :],
                nz_buf.at[slot, pl.ds(h * H, H)],
                nz_sem.at[slot],
            ).wait()

    def out_start(i, slot):
        for h in range(2):
            pltpu.make_async_copy(
                out_buf.at[slot, pl.ds(h * H, H)],
                out_hbm.at[pl.ds(i * CH + h * H, H), :],
                out_sem.at[slot],
            ).start()

    def out_wait(i, slot):
        for h in range(2):
            pltpu.make_async_copy(
                out_buf.at[slot, pl.ds(h * H, H)],
                out_hbm.at[pl.ds(i * CH + h * H, H), :],
                out_sem.at[slot],
            ).wait()

    for s in range(NBUF):
        start_in(s, s)

    a = ac_t_ref[...]
    sa = jnp.sqrt(a)
    sb = jnp.sqrt(jnp.maximum(1.0 - a, 0.0))

    def body(i, carry):
        slot = lax.rem(i, NBUF)
        wait_in(i, slot)

        @pl.when(i >= NBUF)
        def _():
            out_wait(i - NBUF, slot)

        ov = out_buf.at[slot]
        ov[...] = sa * x0_buf[slot] + sb * nz_buf[slot]
        out_start(i, slot)

        @pl.when(i + NBUF < NCHUNK)
        def _():
            start_in(i + NBUF, slot)

        return carry

    lax.fori_loop(0, NCHUNK, body, 0)
    for k in range(NBUF):
        i = NCHUNK - NBUF + k
        out_wait(i, i % NBUF)


def kernel(x_0, timesteps, noise, alphas_cumprod):
    orig_shape = x_0.shape
    x0r = x_0.reshape(BATCH, FEAT).T
    nzr = noise.reshape(BATCH, FEAT).T
    ts = timesteps.astype(jnp.int32)
    ac = alphas_cumprod.astype(jnp.float32)

    ts_pad = jnp.pad(ts, (0, PAD_B - BATCH))
    ac_pad = jnp.pad(ac, (0, PAD_B - BATCH))
    ac_t = _sc_gather(ac_pad, ts_pad)[:BATCH].reshape(1, BATCH)

    out = pl.pallas_call(
        _fma_body,
        in_specs=[
            pl.BlockSpec(memory_space=pltpu.VMEM),
            pl.BlockSpec(memory_space=pl.ANY),
            pl.BlockSpec(memory_space=pl.ANY),
        ],
        out_specs=pl.BlockSpec(memory_space=pl.ANY),
        out_shape=jax.ShapeDtypeStruct((FEAT, BATCH), jnp.float32),
        scratch_shapes=[
            pltpu.VMEM((NBUF, CH, BATCH), jnp.float32),
            pltpu.VMEM((NBUF, CH, BATCH), jnp.float32),
            pltpu.VMEM((NBUF, CH, BATCH), jnp.float32),
            pltpu.SemaphoreType.DMA((NBUF,)),
            pltpu.SemaphoreType.DMA((NBUF,)),
            pltpu.SemaphoreType.DMA((NBUF,)),
        ],
    )(ac_t, x0r, nzr)
    return out.T.reshape(orig_shape)


# speedup vs baseline: 1.0093x; 1.0055x over previous
import functools

import jax
import jax.numpy as jnp
from jax import lax
from jax.experimental import pallas as pl
from jax.experimental.pallas import tpu as pltpu
from jax.experimental.pallas import tpu_sc as plsc

BATCH = 1000
FEAT = 12288
ROWS = 1024

_SC_INFO = plsc.get_sparse_core_info()
_NC = _SC_INFO.num_cores
_NS = _SC_INFO.num_subcores
_NW = _NC * _NS
_PER_W = 40
_ACTIVE = BATCH // _PER_W


def _sc_gather_body(table_hbm, idx_hbm, out_hbm, idx_v, rows_v, sem):
    wid = lax.axis_index("s") * _NC + lax.axis_index("c")

    @pl.when(wid < _ACTIVE)
    def _():
        base = wid * _PER_W
        pltpu.sync_copy(idx_hbm.at[pl.ds(base, _PER_W)], idx_v)
        pltpu.async_copy(table_hbm.at[idx_v], rows_v, sem).wait()
        pltpu.sync_copy(rows_v, out_hbm.at[pl.ds(base, _PER_W)])


_sc_gather = functools.partial(
    pl.kernel,
    out_type=jax.ShapeDtypeStruct((BATCH,), jnp.float32),
    mesh=plsc.VectorSubcoreMesh(core_axis_name="c", subcore_axis_name="s"),
    scratch_types=[
        pltpu.VMEM((_PER_W,), jnp.int32),
        pltpu.VMEM((_PER_W,), jnp.float32),
        pltpu.SemaphoreType.DMA,
    ],
)(_sc_gather_body)


CH = 512
NCHUNK = FEAT // CH
NBUF = 6


def _fma_body(ac_t_ref, x0_hbm, nz_hbm, out_hbm,
              x0_buf, nz_buf, out_buf, x0_sem, nz_sem, out_sem):
    H = CH // 2

    def start_in(i, slot):
        for h in range(2):
            pltpu.make_async_copy(
                x0_hbm.at[pl.ds(i * CH + h * H, H), :],
                x0_buf.at[slot, pl.ds(h * H, H)],
                x0_sem.at[slot],
            ).start()
        for h in range(2):
            pltpu.make_async_copy(
                nz_hbm.at[pl.ds(i * CH + h * H, H), :],
                nz_buf.at[slot, pl.ds(h * H, H)],
                nz_sem.at[slot],
            ).start()

    def wait_in(i, slot):
        for h in range(2):
            pltpu.make_async_copy(
                x0_hbm.at[pl.ds(i * CH + h * H, H), :],
                x0_buf.at[slot, pl.ds(h * H, H)],
                x0_sem.at[slot],
            ).wait()
        for h in range(2):
            pltpu.make_async_copy(
                nz_hbm.at[pl.ds(i * CH + h * H, H), :],
                nz_buf.at[slot, pl.ds(h * H, H)],
                nz_sem.at[slot],
            ).wait()

    def out_start(i, slot):
        for h in range(2):
            pltpu.make_async_copy(
                out_buf.at[slot, pl.ds(h * H, H)],
                out_hbm.at[pl.ds(i * CH + h * H, H), :],
                out_sem.at[slot],
            ).start()

    def out_wait(i, slot):
        for h in range(2):
            pltpu.make_async_copy(
                out_buf.at[slot, pl.ds(h * H, H)],
                out_hbm.at[pl.ds(i * CH + h * H, H), :],
                out_sem.at[slot],
            ).wait()

    for s in range(NBUF):
        start_in(s, s)

    a = ac_t_ref[...]
    sa = jnp.sqrt(a)
    sb = jnp.sqrt(jnp.maximum(1.0 - a, 0.0))

    def body(i, carry):
        slot = lax.rem(i, NBUF)
        wait_in(i, slot)

        @pl.when(i >= NBUF)
        def _():
            out_wait(i - NBUF, slot)

        ov = out_buf.at[slot]
        ov[...] = sa * x0_buf[slot] + sb * nz_buf[slot]
        out_start(i, slot)

        @pl.when(i + NBUF < NCHUNK)
        def _():
            start_in(i + NBUF, slot)

        return carry

    lax.fori_loop(0, NCHUNK, body, 0)
    for k in range(NBUF):
        i = NCHUNK - NBUF + k
        out_wait(i, i % NBUF)


def kernel(x_0, timesteps, noise, alphas_cumprod):
    orig_shape = x_0.shape
    x0r = x_0.reshape(BATCH, FEAT).T
    nzr = noise.reshape(BATCH, FEAT).T
    ts = timesteps.astype(jnp.int32)
    ac = alphas_cumprod.astype(jnp.float32)

    ac_t = _sc_gather(ac, ts).reshape(1, BATCH)

    out = pl.pallas_call(
        _fma_body,
        in_specs=[
            pl.BlockSpec(memory_space=pltpu.VMEM),
            pl.BlockSpec(memory_space=pl.ANY),
            pl.BlockSpec(memory_space=pl.ANY),
        ],
        out_specs=pl.BlockSpec(memory_space=pl.ANY),
        out_shape=jax.ShapeDtypeStruct((FEAT, BATCH), jnp.float32),
        scratch_shapes=[
            pltpu.VMEM((NBUF, CH, BATCH), jnp.float32),
            pltpu.VMEM((NBUF, CH, BATCH), jnp.float32),
            pltpu.VMEM((NBUF, CH, BATCH), jnp.float32),
            pltpu.SemaphoreType.DMA((NBUF,)),
            pltpu.SemaphoreType.DMA((NBUF,)),
            pltpu.SemaphoreType.DMA((NBUF,)),
        ],
    )(ac_t, x0r, nzr)
    return out.T.reshape(orig_shape)
